# baseline (speedup 1.0000x reference)
import functools

import jax
import jax.numpy as jnp
from jax import lax
from jax.experimental import pallas as pl
from jax.experimental.pallas import tpu as pltpu
from jax.experimental.pallas import tpu_sc as plsc

_B = 16384
_V = 1_000_000
_F = 32

_NC = 2
_NS = 16
_NW = _NC * _NS
_BPW = _B // _NW
_CH = 256
_NCHK = _BPW // _CH

_SBR = 256
_NSB = _V // _SBR // 2
_NFULL = (_V // _SBR)
_TAILR = _V - _NFULL * _SBR


def _relayout_body(src3, tail, dst, slab0, slab1, st0, st1, sin, sout):
    wid = lax.axis_index("s") * _NC + lax.axis_index("c")
    lo = wid * _NFULL // _NW
    hi = (wid + 1) * _NFULL // _NW
    iot = lax.iota(jnp.int32, 16)
    svec = [iot >> 3, 2 + (iot >> 3)]
    fvec = iot & 7

    slabs = (slab0, slab1)
    stages = (st0, st1)

    def in_src(sb):
        off = pl.multiple_of(sb * _SBR, _SBR)
        return src3.at[:, :, pl.ds(off, _SBR)]

    def out_dst(sb):
        off = pl.multiple_of(sb * _SBR, _SBR)
        return dst.at[pl.ds(off, _SBR), :]

    @pl.when(lo < hi)
    def _():
        pltpu.async_copy(in_src(lo), slab0, sin)

    @pl.when(lo + 1 < hi)
    def _():
        pltpu.async_copy(in_src(lo + 1), slab1, sin)

    def step(n, carry):
        sb = lo + 2 * n
        for b in range(2):
            @pl.when((sb + b >= lo + 2) & (sb + b < hi + 2))
            def _():
                pltpu.make_async_copy(out_dst(lo), stages[b], sout).wait()

            @pl.when(sb + b < hi)
            def _():
                sbb = sb + b
                pltpu.make_async_copy(in_src(sbb), slabs[b], sin).wait()

                def row(r4, rv):
                    for d in range(4):
                        for h in range(2):
                            v = plsc.load_gather(
                                slabs[b], [svec[h], fvec, rv])
                            stages[b][r4 * 4 + d, pl.ds(16 * h, 16)] = v
                        rv = rv + 1
                    return rv

                lax.fori_loop(0, _SBR // 4, row,
                              jnp.zeros((16,), jnp.int32))
                pltpu.async_copy(stages[b], out_dst(sbb), sout)

                @pl.when(sbb + 2 < hi)
                def _():
                    pltpu.async_copy(in_src(sbb + 2), slabs[b], sin)
        return carry

    nsteps = ((_NFULL + _NW - 1) // _NW + 2 + 1) // 2
    lax.fori_loop(0, nsteps, step, 0)

    @pl.when(wid == 0)
    def _():
        pltpu.sync_copy(tail, dst.at[pl.ds(_NFULL * _SBR, _TAILR), :])


@functools.cache
def _relayout():
    return pl.kernel(
        _relayout_body,
        mesh=plsc.VectorSubcoreMesh(
            core_axis_name="c", subcore_axis_name="s", num_cores=_NC),
        out_type=jax.ShapeDtypeStruct((_V, _F), jnp.float32),
        scratch_types=[
            pltpu.VMEM((4, 8, _SBR), jnp.float32),
            pltpu.VMEM((4, 8, _SBR), jnp.float32),
            pltpu.VMEM((_SBR, _F), jnp.float32),
            pltpu.VMEM((_SBR, _F), jnp.float32),
            pltpu.SemaphoreType.DMA,
            pltpu.SemaphoreType.DMA,
        ],
        compiler_params=pltpu.CompilerParams(needs_layout_passes=False),
    )


def _sc_gather_body(u_e, i_e, u_idx, i_idx, j_idx,
                    out_u, out_i, out_j,
                    idxu, idxi, idxj, su, si, sj, sem):
    wid = lax.axis_index("s") * _NC + lax.axis_index("c")
    base = wid * _BPW
    pltpu.sync_copy(u_idx.at[pl.ds(base, _BPW)], idxu)
    pltpu.sync_copy(i_idx.at[pl.ds(base, _BPW)], idxi)
    pltpu.sync_copy(j_idx.at[pl.ds(base, _BPW)], idxj)

    for c in range(_NCHK):
        def issue(g, carry):
            off = c * _CH + g * 16
            vu = idxu[pl.ds(off, 16)]
            vi = idxi[pl.ds(off, 16)]
            vj = idxj[pl.ds(off, 16)]
            for k in range(16):
                r = g * 16 + k
                pltpu.async_copy(u_e.at[pl.ds(vu[k], 1), :],
                                 su.at[pl.ds(r, 1), :], sem)
                pltpu.async_copy(i_e.at[pl.ds(vi[k], 1), :],
                                 si.at[pl.ds(r, 1), :], sem)
                pltpu.async_copy(i_e.at[pl.ds(vj[k], 1), :],
                                 sj.at[pl.ds(r, 1), :], sem)
            return carry

        lax.fori_loop(0, _CH // 16, issue, 0)

        def drain(r, carry):
            for _ in range(3):
                pltpu.make_async_copy(
                    u_e.at[pl.ds(0, 1), :],
                    su.at[pl.ds(0, 1), :], sem).wait()
            return carry

        lax.fori_loop(0, _CH, drain, 0)
        pltpu.sync_copy(su, out_u.at[pl.ds(base + c * _CH, _CH)])
        pltpu.sync_copy(si, out_i.at[pl.ds(base + c * _CH, _CH)])
        pltpu.sync_copy(sj, out_j.at[pl.ds(base + c * _CH, _CH)])


@functools.cache
def _sc_gather():
    return pl.kernel(
        _sc_gather_body,
        mesh=plsc.VectorSubcoreMesh(
            core_axis_name="c", subcore_axis_name="s", num_cores=_NC),
        out_type=[jax.ShapeDtypeStruct((_B, _F), jnp.float32)] * 3,
        scratch_types=[
            pltpu.VMEM((_BPW,), jnp.int32),
            pltpu.VMEM((_BPW,), jnp.int32),
            pltpu.VMEM((_BPW,), jnp.int32),
            pltpu.VMEM((_CH, _F), jnp.float32),
            pltpu.VMEM((_CH, _F), jnp.float32),
            pltpu.VMEM((_CH, _F), jnp.float32),
            pltpu.SemaphoreType.DMA,
        ],
    )


_BLK = 2048


def _mlp_body(eu, ei, ej, w1u, w1i, b1, w2, b2, w3, b3, wf, bfr, oi, oj):
    hu = jnp.dot(eu[...], w1u[...], preferred_element_type=jnp.float32)

    def head(e_ref, o_ref):
        h = jax.nn.relu(hu + jnp.dot(e_ref[...], w1i[...],
                                     preferred_element_type=jnp.float32)
                        + b1[...])
        h = jax.nn.relu(jnp.dot(h, w2[...],
                                preferred_element_type=jnp.float32) + b2[...])
        h = jax.nn.relu(jnp.dot(h, w3[...],
                                preferred_element_type=jnp.float32) + b3[...])
        o_ref[...] = jnp.sum(h * wf[...], axis=1) + bfr[0, 0]

    head(ei, oi)
    head(ej, oj)


def kernel(user, item_i, item_j, context, uEmbd, iEmbd,
           W1, b1, W2, b2, W3, b3, Wf, bf):
    del context
    user = user.astype(jnp.int32)
    item_i = item_i.astype(jnp.int32)
    item_j = item_j.astype(jnp.int32)
    u_row = _relayout()(uEmbd.T.reshape(4, 8, _V), uEmbd[_NFULL * _SBR:, :])
    i_row = _relayout()(iEmbd.T.reshape(4, 8, _V), iEmbd[_NFULL * _SBR:, :])
    eu, ei, ej = _sc_gather()(u_row, i_row, user, item_i, item_j)

    grid = (_B // _BLK,)
    row_spec = pl.BlockSpec((_BLK, _F), lambda i: (i, 0))
    full2 = lambda shp: pl.BlockSpec(shp, lambda i: (0, 0))
    out_spec = pl.BlockSpec((_BLK,), lambda i: (i,))
    pred_i, pred_j = pl.pallas_call(
        _mlp_body,
        grid=grid,
        in_specs=[row_spec, row_spec, row_spec,
                  full2((_F, 32)), full2((_F, 32)),
                  full2((1, 32)),
                  full2((32, 16)), full2((1, 16)),
                  full2((16, 8)), full2((1, 8)),
                  full2((1, 8)), full2((1, 1))],
        out_specs=[out_spec, out_spec],
        out_shape=[jax.ShapeDtypeStruct((_B,), jnp.float32)] * 2,
    )(eu, ei, ej, W1[:_F, :], W1[_F:, :], b1.reshape(1, 32),
      W2, b2.reshape(1, 16), W3, b3.reshape(1, 8),
      Wf.reshape(1, 8), bf.reshape(1, 1))
    return (pred_i, pred_j)


# speedup vs baseline: 2.1591x; 2.1591x over previous
import functools

import jax
import jax.numpy as jnp
from jax import lax
from jax.experimental import pallas as pl
from jax.experimental.pallas import tpu as pltpu
from jax.experimental.pallas import tpu_sc as plsc

_B = 16384
_V = 1_000_000
_F = 32

_NC = 2
_NS = 16
_NW = _NC * _NS
_BPW = _B // _NW
_CH = 256
_NCHK = _BPW // _CH


def _sc_gather_body(r_u, i_e, u_idx, i_idx, j_idx,
                    out_u, out_i, out_j,
                    idxu, idxi, idxj, su, si, sj, sem):
    wid = lax.axis_index("s") * _NC + lax.axis_index("c")
    base = wid * _BPW
    pltpu.sync_copy(u_idx.at[pl.ds(base, _BPW)], idxu)
    pltpu.sync_copy(i_idx.at[pl.ds(base, _BPW)], idxi)
    pltpu.sync_copy(j_idx.at[pl.ds(base, _BPW)], idxj)

    for c in range(_NCHK):
        def issue(g, carry):
            off = c * _CH + g * 16
            gu = idxu[pl.ds(off, 16)] >> 2
            vi = idxi[pl.ds(off, 16)]
            vj = idxj[pl.ds(off, 16)]
            for k in range(16):
                r = g * 16 + k
                pltpu.async_copy(r_u.at[pl.ds(gu[k], 1), :],
                                 su.at[pl.ds(r, 1), :], sem)
                pltpu.async_copy(i_e.at[pl.ds(vi[k], 1), :],
                                 si.at[pl.ds(r, 1), :], sem)
                pltpu.async_copy(i_e.at[pl.ds(vj[k], 1), :],
                                 sj.at[pl.ds(r, 1), :], sem)
            return carry

        lax.fori_loop(0, _CH // 16, issue, 0)

        def drain(r, carry):
            pltpu.make_async_copy(
                r_u.at[pl.ds(0, 1), :],
                su.at[pl.ds(0, 1), :], sem).wait()
            for _ in range(2):
                pltpu.make_async_copy(
                    i_e.at[pl.ds(0, 1), :],
                    si.at[pl.ds(0, 1), :], sem).wait()
            return carry

        lax.fori_loop(0, _CH, drain, 0)
        pltpu.sync_copy(su, out_u.at[pl.ds(base + c * _CH, _CH)])
        pltpu.sync_copy(si, out_i.at[pl.ds(base + c * _CH, _CH)])
        pltpu.sync_copy(sj, out_j.at[pl.ds(base + c * _CH, _CH)])


@functools.cache
def _sc_gather():
    return pl.kernel(
        _sc_gather_body,
        mesh=plsc.VectorSubcoreMesh(
            core_axis_name="c", subcore_axis_name="s", num_cores=_NC),
        out_type=[jax.ShapeDtypeStruct((_B, 128), jnp.float32),
                  jax.ShapeDtypeStruct((_B, _F), jnp.float32),
                  jax.ShapeDtypeStruct((_B, _F), jnp.float32)],
        scratch_types=[
            pltpu.VMEM((_BPW,), jnp.int32),
            pltpu.VMEM((_BPW,), jnp.int32),
            pltpu.VMEM((_BPW,), jnp.int32),
            pltpu.VMEM((_CH, 128), jnp.float32),
            pltpu.VMEM((_CH, _F), jnp.float32),
            pltpu.VMEM((_CH, _F), jnp.float32),
            pltpu.SemaphoreType.DMA,
        ],
    )


_BLK = 2048


def _mlp_body(su, ei, ej, ui, w1u, w1i, b1, w2, b2, w3, b3, wf, bfr, oi, oj):
    a = ui[...] & 3
    slab = su[...]
    eu = jnp.zeros((_BLK, _F), jnp.float32)
    for q in range(4):
        eu += slab[:, _F * q:_F * (q + 1)] * (a == q).astype(jnp.float32)
    hu = jnp.dot(eu, w1u[...], preferred_element_type=jnp.float32)

    def head(e_ref, o_ref):
        h = jax.nn.relu(hu + jnp.dot(e_ref[...], w1i[...],
                                     preferred_element_type=jnp.float32)
                        + b1[...])
        h = jax.nn.relu(jnp.dot(h, w2[...],
                                preferred_element_type=jnp.float32) + b2[...])
        h = jax.nn.relu(jnp.dot(h, w3[...],
                                preferred_element_type=jnp.float32) + b3[...])
        o_ref[...] = jnp.sum(h * wf[...], axis=1) + bfr[0, 0]

    head(ei, oi)
    head(ej, oj)


def kernel(user, item_i, item_j, context, uEmbd, iEmbd,
           W1, b1, W2, b2, W3, b3, Wf, bf):
    del context
    user = user.astype(jnp.int32)
    item_i = item_i.astype(jnp.int32)
    item_j = item_j.astype(jnp.int32)
    r_u = uEmbd.reshape(_V // 4, 128)
    su, ei, ej = _sc_gather()(r_u, iEmbd, user, item_i, item_j)

    grid = (_B // _BLK,)
    row_spec = pl.BlockSpec((_BLK, _F), lambda i: (i, 0))
    full2 = lambda shp: pl.BlockSpec(shp, lambda i: (0, 0))
    out_spec = pl.BlockSpec((_BLK,), lambda i: (i,))
    pred_i, pred_j = pl.pallas_call(
        _mlp_body,
        grid=grid,
        in_specs=[pl.BlockSpec((_BLK, 128), lambda i: (i, 0)),
                  row_spec, row_spec,
                  pl.BlockSpec((_BLK, 1), lambda i: (i, 0)),
                  full2((_F, 32)), full2((_F, 32)),
                  full2((1, 32)),
                  full2((32, 16)), full2((1, 16)),
                  full2((16, 8)), full2((1, 8)),
                  full2((1, 8)), full2((1, 1))],
        out_specs=[out_spec, out_spec],
        out_shape=[jax.ShapeDtypeStruct((_B,), jnp.float32)] * 2,
    )(su, ei, ej, user.reshape(_B, 1),
      W1[:_F, :], W1[_F:, :], b1.reshape(1, 32),
      W2, b2.reshape(1, 16), W3, b3.reshape(1, 8),
      Wf.reshape(1, 8), bf.reshape(1, 1))
    return (pred_i, pred_j)


# speedup vs baseline: 2.4688x; 1.1435x over previous
import functools

import jax
import jax.numpy as jnp
from jax import lax
from jax.experimental import pallas as pl
from jax.experimental.pallas import tpu as pltpu
from jax.experimental.pallas import tpu_sc as plsc

_B = 16384
_F = 32

_NC = 2
_NS = 16
_NW = _NC * _NS
_BPW = _B // _NW
_CH = 256
_NCHK = _BPW // _CH


def _sc_gather_body(u_e, i_e, u_idx, i_idx, j_idx,
                    out_u, out_i, out_j,
                    idxu, idxi, idxj, su, si, sj, sem):
    wid = lax.axis_index("s") * _NC + lax.axis_index("c")
    base = wid * _BPW
    pltpu.sync_copy(u_idx.at[pl.ds(base, _BPW)], idxu)
    pltpu.sync_copy(i_idx.at[pl.ds(base, _BPW)], idxi)
    pltpu.sync_copy(j_idx.at[pl.ds(base, _BPW)], idxj)

    for c in range(_NCHK):
        def issue(g, carry):
            off = c * _CH + g * 16
            vu = idxu[pl.ds(off, 16)]
            vi = idxi[pl.ds(off, 16)]
            vj = idxj[pl.ds(off, 16)]
            for k in range(16):
                r = g * 16 + k
                pltpu.async_copy(u_e.at[pl.ds(vu[k], 1), :],
                                 su.at[pl.ds(r, 1), :], sem)
                pltpu.async_copy(i_e.at[pl.ds(vi[k], 1), :],
                                 si.at[pl.ds(r, 1), :], sem)
                pltpu.async_copy(i_e.at[pl.ds(vj[k], 1), :],
                                 sj.at[pl.ds(r, 1), :], sem)
            return carry

        lax.fori_loop(0, _CH // 16, issue, 0)

        def drain(r, carry):
            for _ in range(3):
                pltpu.make_async_copy(
                    u_e.at[pl.ds(0, 1), :],
                    su.at[pl.ds(0, 1), :], sem).wait()
            return carry

        lax.fori_loop(0, _CH, drain, 0)
        pltpu.sync_copy(su, out_u.at[pl.ds(base + c * _CH, _CH)])
        pltpu.sync_copy(si, out_i.at[pl.ds(base + c * _CH, _CH)])
        pltpu.sync_copy(sj, out_j.at[pl.ds(base + c * _CH, _CH)])


@functools.cache
def _sc_gather():
    return pl.kernel(
        _sc_gather_body,
        mesh=plsc.VectorSubcoreMesh(
            core_axis_name="c", subcore_axis_name="s", num_cores=_NC),
        out_type=[jax.ShapeDtypeStruct((_B, _F), jnp.float32)] * 3,
        scratch_types=[
            pltpu.VMEM((_BPW,), jnp.int32),
            pltpu.VMEM((_BPW,), jnp.int32),
            pltpu.VMEM((_BPW,), jnp.int32),
            pltpu.VMEM((_CH, _F), jnp.float32),
            pltpu.VMEM((_CH, _F), jnp.float32),
            pltpu.VMEM((_CH, _F), jnp.float32),
            pltpu.SemaphoreType.DMA,
        ],
    )


_BLK = 2048


def _mlp_body(eu, ei, ej, w1u, w1i, b1, w2, b2, w3, b3, wf, bfr, oi, oj):
    hu = jnp.dot(eu[...], w1u[...], preferred_element_type=jnp.float32)

    def head(e_ref, o_ref):
        h = jax.nn.relu(hu + jnp.dot(e_ref[...], w1i[...],
                                     preferred_element_type=jnp.float32)
                        + b1[...])
        h = jax.nn.relu(jnp.dot(h, w2[...],
                                preferred_element_type=jnp.float32) + b2[...])
        h = jax.nn.relu(jnp.dot(h, w3[...],
                                preferred_element_type=jnp.float32) + b3[...])
        o_ref[...] = jnp.sum(h * wf[...], axis=1) + bfr[0, 0]

    head(ei, oi)
    head(ej, oj)


def kernel(user, item_i, item_j, context, uEmbd, iEmbd,
           W1, b1, W2, b2, W3, b3, Wf, bf):
    del context
    eu, ei, ej = _sc_gather()(uEmbd, iEmbd,
                              user.astype(jnp.int32),
                              item_i.astype(jnp.int32),
                              item_j.astype(jnp.int32))

    grid = (_B // _BLK,)
    row_spec = pl.BlockSpec((_BLK, _F), lambda i: (i, 0))
    full2 = lambda shp: pl.BlockSpec(shp, lambda i: (0, 0))
    out_spec = pl.BlockSpec((_BLK,), lambda i: (i,))
    pred_i, pred_j = pl.pallas_call(
        _mlp_body,
        grid=grid,
        in_specs=[row_spec, row_spec, row_spec,
                  full2((_F, 32)), full2((_F, 32)),
                  full2((1, 32)),
                  full2((32, 16)), full2((1, 16)),
                  full2((16, 8)), full2((1, 8)),
                  full2((1, 8)), full2((1, 1))],
        out_specs=[out_spec, out_spec],
        out_shape=[jax.ShapeDtypeStruct((_B,), jnp.float32)] * 2,
    )(eu, ei, ej, W1[:_F, :], W1[_F:, :], b1.reshape(1, 32),
      W2, b2.reshape(1, 16), W3, b3.reshape(1, 8),
      Wf.reshape(1, 8), bf.reshape(1, 1))
    return (pred_i, pred_j)
